# baseline (device time: 16214 ns/iter reference)
import jax
import jax.numpy as jnp
from jax import lax
from jax.experimental import pallas as pl
from jax.experimental.pallas import tpu as pltpu

N_DEV = 8
K = 2
M_CH = 64
SUB = M_CH // K


def kernel(t):
    m, n = t.shape

    def body(x_ref, out_ref, red_ref, fbuf_ref,
             s1_send, s1_recv, s2_send, s2_recv):
        i = lax.axis_index("i")

        barrier_sem = pltpu.get_barrier_semaphore()
        for o in range(1, N_DEV):
            pl.semaphore_signal(
                barrier_sem, inc=1,
                device_id=((i + o) % N_DEV,),
                device_id_type=pl.DeviceIdType.MESH,
            )
        pl.semaphore_wait(barrier_sem, N_DEV - 1)

        ph1 = {}
        for k in range(K):
            for o in range(1, N_DEV):
                j = (i + o) % N_DEV
                rdma = pltpu.make_async_remote_copy(
                    src_ref=x_ref.at[pl.ds(j * M_CH + k * SUB, SUB), :],
                    dst_ref=red_ref.at[o, pl.ds(k * SUB, SUB), :],
                    send_sem=s1_send.at[o - 1, k],
                    recv_sem=s1_recv.at[o - 1, k],
                    device_id=(j,),
                    device_id_type=pl.DeviceIdType.MESH,
                )
                rdma.start()
                ph1[(o, k)] = rdma

        ph2 = {}
        for k in range(K):
            acc = x_ref[pl.ds(i * M_CH + k * SUB, SUB), :]
            for o in range(1, N_DEV):
                ph1[(o, k)].wait_recv()
                acc = acc + red_ref[o, pl.ds(k * SUB, SUB), :]
            r = jnp.maximum(acc, 0.0)
            fv = jnp.tanh(acc) * acc * acc + r * r * r
            fbuf_ref[pl.ds(k * SUB, SUB), :] = fv
            out_ref[pl.ds(i * M_CH + k * SUB, SUB), :] = fv
            for o in range(1, N_DEV):
                j = (i + o) % N_DEV
                rdma = pltpu.make_async_remote_copy(
                    src_ref=fbuf_ref.at[pl.ds(k * SUB, SUB), :],
                    dst_ref=out_ref.at[pl.ds(i * M_CH + k * SUB, SUB), :],
                    send_sem=s2_send.at[o - 1, k],
                    recv_sem=s2_recv.at[o - 1, k],
                    device_id=(j,),
                    device_id_type=pl.DeviceIdType.MESH,
                )
                rdma.start()
                ph2[(o, k)] = rdma

        for k in range(K):
            for o in range(1, N_DEV):
                ph2[(o, k)].wait_recv()
        for k in range(K):
            for o in range(1, N_DEV):
                ph1[(o, k)].wait_send()
                ph2[(o, k)].wait_send()

    return pl.pallas_call(
        body,
        out_shape=jax.ShapeDtypeStruct((m, n), jnp.float32),
        in_specs=[pl.BlockSpec(memory_space=pltpu.VMEM)],
        out_specs=pl.BlockSpec(memory_space=pltpu.VMEM),
        scratch_shapes=[
            pltpu.VMEM((N_DEV, M_CH, n), jnp.float32),
            pltpu.VMEM((M_CH, n), jnp.float32),
            pltpu.SemaphoreType.DMA((N_DEV - 1, K)),
            pltpu.SemaphoreType.DMA((N_DEV - 1, K)),
            pltpu.SemaphoreType.DMA((N_DEV - 1, K)),
            pltpu.SemaphoreType.DMA((N_DEV - 1, K)),
        ],
        compiler_params=pltpu.CompilerParams(collective_id=0),
    )(t)


# device time: 14317 ns/iter; 1.1325x vs baseline; 1.1325x over previous
import jax
import jax.numpy as jnp
from jax import lax
from jax.experimental import pallas as pl
from jax.experimental.pallas import tpu as pltpu

N_DEV = 8
N_STAGES = 3
ROW_SPLIT = (64, 56, 56, 56, 56, 56, 56, 56, 56)
N_CHUNKS = len(ROW_SPLIT)


def kernel(t):
    m, n = t.shape
    row_off = [sum(ROW_SPLIT[:c]) for c in range(N_CHUNKS)]

    def body(x_ref, out_ref, acc_ref, comm_ref, send_sems, recv_sems):
        i = lax.axis_index("i")

        px = i + 1 - 2 * (i % 2)
        base = (i // 4) * 4
        py = base + 3 - (i - base)
        pz = (i + 4) % N_DEV
        dims = [px, py, pz]

        barrier_sem = pltpu.get_barrier_semaphore()
        for p in dims:
            pl.semaphore_signal(
                barrier_sem, inc=1,
                device_id=(p,), device_id_type=pl.DeviceIdType.MESH,
            )
        pl.semaphore_wait(barrier_sem, 3)

        def make_rdma(c, s):
            r0, rc = row_off[c], ROW_SPLIT[c]
            src = x_ref if s == 0 else acc_ref
            return pltpu.make_async_remote_copy(
                src_ref=src.at[pl.ds(r0, rc), :],
                dst_ref=comm_ref.at[s, pl.ds(r0, rc), :],
                send_sem=send_sems.at[c, s],
                recv_sem=recv_sems.at[c, s],
                device_id=(dims[(s + c) % 3],),
                device_id_type=pl.DeviceIdType.MESH,
            )

        rdmas = [[None] * N_STAGES for _ in range(N_CHUNKS)]
        for c in range(N_CHUNKS):
            rdmas[c][0] = make_rdma(c, 0)
            rdmas[c][0].start()

        for s in range(N_STAGES):
            for c in range(N_CHUNKS):
                r0, rc = row_off[c], ROW_SPLIT[c]
                rdmas[c][s].wait()
                prev = x_ref if s == 0 else acc_ref
                acc_ref[pl.ds(r0, rc), :] = (
                    prev[pl.ds(r0, rc), :] + comm_ref[s, pl.ds(r0, rc), :]
                )
                if s + 1 < N_STAGES:
                    rdmas[c][s + 1] = make_rdma(c, s + 1)
                    rdmas[c][s + 1].start()
                else:
                    sv = acc_ref[pl.ds(r0, rc), :]
                    r = jnp.maximum(sv, 0.0)
                    out_ref[pl.ds(r0, rc), :] = (
                        jnp.tanh(sv) * sv * sv + r * r * r
                    )

    return pl.pallas_call(
        body,
        out_shape=jax.ShapeDtypeStruct((m, n), jnp.float32),
        in_specs=[pl.BlockSpec(memory_space=pltpu.VMEM)],
        out_specs=pl.BlockSpec(memory_space=pltpu.VMEM),
        scratch_shapes=[
            pltpu.VMEM((m, n), jnp.float32),
            pltpu.VMEM((N_STAGES, m, n), jnp.float32),
            pltpu.SemaphoreType.DMA((N_CHUNKS, N_STAGES)),
            pltpu.SemaphoreType.DMA((N_CHUNKS, N_STAGES)),
        ],
        compiler_params=pltpu.CompilerParams(collective_id=0),
    )(t)


# device time: 14186 ns/iter; 1.1430x vs baseline; 1.0092x over previous
import jax
import jax.numpy as jnp
from jax import lax
from jax.experimental import pallas as pl
from jax.experimental.pallas import tpu as pltpu

N_DEV = 8
N_STAGES = 3
ROW_SPLIT = (48, 48, 48, 48, 40, 40, 40, 40, 40, 40, 40, 40)
N_CHUNKS = len(ROW_SPLIT)


def kernel(t):
    m, n = t.shape
    row_off = [sum(ROW_SPLIT[:c]) for c in range(N_CHUNKS)]

    def body(x_ref, out_ref, acc_ref, comm_ref, send_sems, recv_sems):
        i = lax.axis_index("i")

        px = i + 1 - 2 * (i % 2)
        base = (i // 4) * 4
        py = base + 3 - (i - base)
        pz = (i + 4) % N_DEV
        dims = [px, py, pz]

        barrier_sem = pltpu.get_barrier_semaphore()
        for p in dims:
            pl.semaphore_signal(
                barrier_sem, inc=1,
                device_id=(p,), device_id_type=pl.DeviceIdType.MESH,
            )
        pl.semaphore_wait(barrier_sem, 3)

        def make_rdma(c, s):
            r0, rc = row_off[c], ROW_SPLIT[c]
            src = x_ref if s == 0 else acc_ref
            return pltpu.make_async_remote_copy(
                src_ref=src.at[pl.ds(r0, rc), :],
                dst_ref=comm_ref.at[s, pl.ds(r0, rc), :],
                send_sem=send_sems.at[c, s],
                recv_sem=recv_sems.at[c, s],
                device_id=(dims[(s + c) % 3],),
                device_id_type=pl.DeviceIdType.MESH,
            )

        rdmas = [[None] * N_STAGES for _ in range(N_CHUNKS)]
        for c in range(N_CHUNKS):
            rdmas[c][0] = make_rdma(c, 0)
            rdmas[c][0].start()

        for s in range(N_STAGES):
            for c in range(N_CHUNKS):
                r0, rc = row_off[c], ROW_SPLIT[c]
                rdmas[c][s].wait()
                prev = x_ref if s == 0 else acc_ref
                sv = prev[pl.ds(r0, rc), :] + comm_ref[s, pl.ds(r0, rc), :]
                if s + 1 < N_STAGES:
                    acc_ref[pl.ds(r0, rc), :] = sv
                    rdmas[c][s + 1] = make_rdma(c, s + 1)
                    rdmas[c][s + 1].start()
                else:
                    r = jnp.maximum(sv, 0.0)
                    out_ref[pl.ds(r0, rc), :] = (
                        jnp.tanh(sv) * sv * sv + r * r * r
                    )

    return pl.pallas_call(
        body,
        out_shape=jax.ShapeDtypeStruct((m, n), jnp.float32),
        in_specs=[pl.BlockSpec(memory_space=pltpu.VMEM)],
        out_specs=pl.BlockSpec(memory_space=pltpu.VMEM),
        scratch_shapes=[
            pltpu.VMEM((m, n), jnp.float32),
            pltpu.VMEM((N_STAGES, m, n), jnp.float32),
            pltpu.SemaphoreType.DMA((N_CHUNKS, N_STAGES)),
            pltpu.SemaphoreType.DMA((N_CHUNKS, N_STAGES)),
        ],
        compiler_params=pltpu.CompilerParams(collective_id=0),
    )(t)
